# Initial kernel scaffold; baseline (speedup 1.0000x reference)
import jax
import jax.numpy as jnp
from jax.experimental import pallas as pl


def kernel(x, edge_index_0, edge_index_1, W1, W2, bias):
    raise NotImplementedError("write your pallas kernel here")


# speedup vs baseline: 3.7058x; 3.7058x over previous
import functools

import jax
import jax.numpy as jnp
from jax import lax
from jax.experimental import pallas as pl
from jax.experimental.pallas import tpu as pltpu
from jax.experimental.pallas import tpu_sc as plsc

N = 10000
L = 2
F = 128
H = 64
M = L * N
E2 = 640000
MP = 20480
NTILES = 16
CH = 80


def _sc_hist(src, trg, zeros_m):
    ept = E2 // (2 * NTILES)
    n_chunks = ept // CH
    ms = MP // NTILES
    mesh = plsc.VectorSubcoreMesh(core_axis_name="c", subcore_axis_name="s")

    @functools.partial(
        pl.kernel,
        out_type=jax.ShapeDtypeStruct((2, 2, MP), jnp.float32),
        mesh=mesh,
        scratch_types=[
            pltpu.VMEM((CH,), jnp.int32),
            pltpu.VMEM((CH,), jnp.float32),
            pltpu.VMEM_SHARED((MP,), jnp.float32),
            pltpu.VMEM_SHARED((MP,), jnp.float32),
            pltpu.SemaphoreType.DMA,
        ],
    )
    def hist_kernel(src_hbm, trg_hbm, zeros_hbm, out_hbm,
                    idx_v, ones_v, acc_s, acc_t, sem):
        c = lax.axis_index("c")
        s = lax.axis_index("s")
        for j in range(CH // 16):
            ones_v[pl.ds(j * 16, 16)] = jnp.full((16,), 1.0, jnp.float32)
        pltpu.sync_copy(zeros_hbm.at[pl.ds(s * ms, ms)], acc_s.at[pl.ds(s * ms, ms)])
        pltpu.sync_copy(zeros_hbm.at[pl.ds(s * ms, ms)], acc_t.at[pl.ds(s * ms, ms)])
        plsc.subcore_barrier()
        base = (c * NTILES + s) * ept

        def body(i, carry):
            off = base + i * CH
            pltpu.sync_copy(src_hbm.at[pl.ds(off, CH)], idx_v)
            pltpu.sync_copy(ones_v, acc_s.at[idx_v], add=True)
            pltpu.sync_copy(trg_hbm.at[pl.ds(off, CH)], idx_v)
            pltpu.sync_copy(ones_v, acc_t.at[idx_v], add=True)
            return carry

        lax.fori_loop(0, n_chunks, body, 0)
        plsc.subcore_barrier()
        pltpu.sync_copy(acc_s.at[pl.ds(s * ms, ms)], out_hbm.at[c, 0, pl.ds(s * ms, ms)])
        pltpu.sync_copy(acc_t.at[pl.ds(s * ms, ms)], out_hbm.at[c, 1, pl.ds(s * ms, ms)])

    return hist_kernel(src, trg, zeros_m)


def _sc_main(table, src, trg, zeros_mh):
    ept = E2 // NTILES
    n_chunks = ept // CH
    ms = MP // NTILES
    mesh = plsc.VectorSubcoreMesh(core_axis_name="c", subcore_axis_name="s")

    @functools.partial(
        pl.kernel,
        out_type=jax.ShapeDtypeStruct((2, MP, H), jnp.float32),
        mesh=mesh,
        scratch_types=[
            pltpu.VMEM((CH,), jnp.int32),
            pltpu.VMEM((CH,), jnp.int32),
            pltpu.VMEM((CH, H), jnp.float32),
            pltpu.VMEM_SHARED((MP, H), jnp.float32),
            pltpu.SemaphoreType.DMA,
        ],
        compiler_params=pltpu.CompilerParams(use_tc_tiling_on_sc=False),
    )
    def main_kernel(table_hbm, src_hbm, trg_hbm, zeros_hbm, out_hbm,
                    idx_s, idx_t, rows, acc, sem):
        c = lax.axis_index("c")
        s = lax.axis_index("s")
        pltpu.sync_copy(zeros_hbm.at[pl.ds(s * ms, ms)], acc.at[pl.ds(s * ms, ms)])
        plsc.subcore_barrier()
        cm = c * M
        base = s * ept

        def body(i, carry):
            off = base + i * CH
            pltpu.sync_copy(src_hbm.at[pl.ds(off, CH)], idx_s)
            for j in range(CH // 16):
                sl = pl.ds(j * 16, 16)
                idx_s[sl] = idx_s[sl] + cm
            pltpu.sync_copy(trg_hbm.at[pl.ds(off, CH)], idx_t)
            pltpu.async_copy(table_hbm.at[idx_s], rows, sem).wait()
            pltpu.sync_copy(rows, acc.at[idx_t], add=True)
            return carry

        lax.fori_loop(0, n_chunks, body, 0)
        plsc.subcore_barrier()
        pltpu.sync_copy(acc.at[pl.ds(s * ms, ms)], out_hbm.at[c, pl.ds(s * ms, ms)])

    return main_kernel(table, src, trg, zeros_mh)


_BN = 1000


def _tc_a(x, w1, degs_t):
    def body(x_ref, w1_ref, deg_ref, table_ref, d_ref, innorm_ref):
        dg = deg_ref[...]
        on0 = lax.rsqrt(dg[:, 0] + dg[:, 4] + 2.0)
        on1 = lax.rsqrt(dg[:, 1] + dg[:, 5] + 2.0)
        in0 = lax.rsqrt(dg[:, 2] + dg[:, 6] + 2.0)
        in1 = lax.rsqrt(dg[:, 3] + dg[:, 7] + 2.0)
        innorm_ref[...] = jnp.stack([in0, in1], axis=-1)
        xb = x_ref[0]
        p = lax.dot_general(xb, w1_ref[...],
                            (((1,), (1,)), ((), ())),
                            preferred_element_type=jnp.float32)
        p0 = p[:, :F] * on0[:, None]
        p1 = p[:, F:] * on1[:, None]
        d_ref[...] = p0 + p1
        table_ref[0] = p0[:, :H]
        table_ref[1] = p1[:, :H]
        table_ref[2] = p0[:, H:]
        table_ref[3] = p1[:, H:]

    return pl.pallas_call(
        body,
        grid=(N // _BN,),
        in_specs=[
            pl.BlockSpec((1, _BN, F), lambda i: (0, i, 0)),
            pl.BlockSpec((2 * F, F), lambda i: (0, 0)),
            pl.BlockSpec((_BN, 8), lambda i: (i, 0)),
        ],
        out_specs=[
            pl.BlockSpec((4, _BN, H), lambda i: (0, i, 0)),
            pl.BlockSpec((_BN, F), lambda i: (i, 0)),
            pl.BlockSpec((_BN, 2), lambda i: (i, 0)),
        ],
        out_shape=[
            jax.ShapeDtypeStruct((4, N, H), jnp.float32),
            jax.ShapeDtypeStruct((N, F), jnp.float32),
            jax.ShapeDtypeStruct((N, 2), jnp.float32),
        ],
    )(x, w1, degs_t)


def _tc_b(aggsc, d, innorm, bias, w2):
    def body(agg_ref, d_ref, innorm_ref, bias_ref, w2_ref, out_ref):
        agg = agg_ref[...]
        dv = d_ref[...]
        a0 = jnp.concatenate([agg[0, 0], agg[1, 0]], axis=-1) + dv
        a1 = jnp.concatenate([agg[0, 1], agg[1, 1]], axis=-1) + dv
        n0 = innorm_ref[:, 0]
        n1 = innorm_ref[:, 1]
        a0 = a0 * n0[:, None] + bias_ref[0][None, :]
        a1 = a1 * n1[:, None] + bias_ref[1][None, :]
        a0 = jnp.where(a0 >= 0, a0, 0.2 * a0)
        a1 = jnp.where(a1 >= 0, a1, 0.2 * a1)
        y = jnp.concatenate([a0, a1], axis=-1)
        out_ref[0] = lax.dot_general(y, w2_ref[...],
                                     (((1,), (1,)), ((), ())),
                                     preferred_element_type=jnp.float32)

    return pl.pallas_call(
        body,
        grid=(N // _BN,),
        in_specs=[
            pl.BlockSpec((2, 2, _BN, H), lambda i: (0, 0, i, 0)),
            pl.BlockSpec((_BN, F), lambda i: (i, 0)),
            pl.BlockSpec((_BN, 2), lambda i: (i, 0)),
            pl.BlockSpec((2, F), lambda i: (0, 0)),
            pl.BlockSpec((F, 2 * F), lambda i: (0, 0)),
        ],
        out_specs=pl.BlockSpec((1, _BN, F), lambda i: (0, i, 0)),
        out_shape=jax.ShapeDtypeStruct((1, N, F), jnp.float32),
    )(aggsc, d, innorm, bias, w2)


def kernel(x, edge_index_0, edge_index_1, W1, W2, bias):
    src = jnp.concatenate([edge_index_0[0], edge_index_1[0] + N])
    trg = jnp.concatenate([edge_index_0[1], edge_index_1[1] + N])
    zeros_m = jnp.zeros((MP,), jnp.float32)
    zeros_mh = jnp.zeros((MP, H), jnp.float32)
    degs = _sc_hist(src, trg, zeros_m)[:, :, :M]
    degs_t = degs.reshape(2, 2, 2, N).transpose(3, 0, 1, 2).reshape(N, 8)
    table4, d, innorm = _tc_a(x, W1, degs_t)
    aggsc = _sc_main(table4.reshape(2 * M, H), src, trg, zeros_mh)[:, :M]
    return _tc_b(aggsc.reshape(2, 2, N, H), d, innorm, bias, W2)


# speedup vs baseline: 12.4647x; 3.3635x over previous
import functools

import jax
import jax.numpy as jnp
from jax import lax
from jax.experimental import pallas as pl
from jax.experimental.pallas import tpu as pltpu
from jax.experimental.pallas import tpu_sc as plsc

N = 10000
L = 2
F = 128
H = 64
M = L * N
E2 = 640000
MP = 20480
NTILES = 16
CH = 80


_NCKH = E2 // (2 * NTILES) // CH
_KH = 8


def _sc_hist(src_h, trg_h, zeros_m):
    ms = MP // NTILES
    mesh = plsc.VectorSubcoreMesh(core_axis_name="c", subcore_axis_name="s")

    @functools.partial(
        pl.kernel,
        out_type=jax.ShapeDtypeStruct((2, 2, MP), jnp.float32),
        mesh=mesh,
        scratch_types=[
            pltpu.VMEM((_NCKH, CH), jnp.int32),
            pltpu.VMEM((_NCKH, CH), jnp.int32),
            pltpu.VMEM((CH,), jnp.float32),
            pltpu.VMEM_SHARED((MP,), jnp.float32),
            pltpu.VMEM_SHARED((MP,), jnp.float32),
            pltpu.SemaphoreType.DMA,
            pltpu.SemaphoreType.DMA,
            pltpu.SemaphoreType.DMA,
        ],
    )
    def hist_kernel(src_hbm, trg_hbm, zeros_hbm, out_hbm,
                    idx_s, idx_t, ones_v, acc_s, acc_t, sem_i, sem_a, sem_b):
        c = lax.axis_index("c")
        s = lax.axis_index("s")
        for j in range(CH // 16):
            ones_v[pl.ds(j * 16, 16)] = jnp.full((16,), 1.0, jnp.float32)
        d1 = pltpu.make_async_copy(src_hbm.at[c, s], idx_s, sem_i)
        d2 = pltpu.make_async_copy(trg_hbm.at[c, s], idx_t, sem_i)
        d1.start()
        d2.start()
        pltpu.sync_copy(zeros_hbm.at[pl.ds(s * ms, ms)], acc_s.at[pl.ds(s * ms, ms)])
        pltpu.sync_copy(zeros_hbm.at[pl.ds(s * ms, ms)], acc_t.at[pl.ds(s * ms, ms)])
        d1.wait()
        d2.wait()
        plsc.subcore_barrier()

        def fire(g):
            pltpu.make_async_copy(ones_v, acc_s.at[idx_s.at[g]], sem_a).start(add=True)
            pltpu.make_async_copy(ones_v, acc_t.at[idx_t.at[g]], sem_b).start(add=True)

        def drain(g):
            pltpu.make_async_copy(ones_v, acc_s.at[idx_s.at[g]], sem_a).wait()
            pltpu.make_async_copy(ones_v, acc_t.at[idx_t.at[g]], sem_b).wait()

        for g in range(_KH):
            fire(g)

        def body(i, carry):
            drain(i - _KH)
            fire(i)
            return carry

        lax.fori_loop(_KH, _NCKH, body, 0)
        for g in range(_KH):
            drain(g)
        plsc.subcore_barrier()
        pltpu.sync_copy(acc_s.at[pl.ds(s * ms, ms)], out_hbm.at[c, 0, pl.ds(s * ms, ms)])
        pltpu.sync_copy(acc_t.at[pl.ds(s * ms, ms)], out_hbm.at[c, 1, pl.ds(s * ms, ms)])

    return hist_kernel(src_h, trg_h, zeros_m)


_NCK = E2 // NTILES // CH
_NBUF = 5
_IBLK = 50
_NBLK = _NCK // _IBLK
_NGRPB = _IBLK // _NBUF


def _sc_main(table, srcs_m, trg_m, zeros_mh):
    ms = MP // NTILES
    mesh = plsc.VectorSubcoreMesh(core_axis_name="c", subcore_axis_name="s")

    @functools.partial(
        pl.kernel,
        out_type=jax.ShapeDtypeStruct((2, MP, H), jnp.float32),
        mesh=mesh,
        scratch_types=[
            pltpu.VMEM((2, _IBLK, CH), jnp.int32),
            pltpu.VMEM((2, _IBLK, CH), jnp.int32),
            pltpu.VMEM((_NBUF, CH, H), jnp.float32),
            pltpu.VMEM_SHARED((MP, H), jnp.float32),
            pltpu.SemaphoreType.DMA((2,)),
            pltpu.SemaphoreType.DMA((_NBUF,)),
            pltpu.SemaphoreType.DMA((_NBUF,)),
        ],
        compiler_params=pltpu.CompilerParams(use_tc_tiling_on_sc=False),
    )
    def main_kernel(table_hbm, src_hbm, trg_hbm, zeros_hbm, out_hbm,
                    isb, itb, rows, acc, sem_ib, sem_g, sem_sc):
        c = lax.axis_index("c")
        s = lax.axis_index("s")

        def idx_load(blk, parity):
            return (pltpu.make_async_copy(src_hbm.at[c, s, blk],
                                          isb.at[parity], sem_ib.at[parity]),
                    pltpu.make_async_copy(trg_hbm.at[s, blk],
                                          itb.at[parity], sem_ib.at[parity]))

        for d in idx_load(0, 0):
            d.start()
        pltpu.sync_copy(zeros_hbm.at[pl.ds(s * ms, ms)], acc.at[pl.ds(s * ms, ms)])
        plsc.subcore_barrier()

        def gather(parity, k, b):
            return pltpu.make_async_copy(table_hbm.at[isb.at[parity, k]],
                                         rows.at[b], sem_g.at[b])

        def scatter(parity, k, b):
            return pltpu.make_async_copy(rows.at[b],
                                         acc.at[itb.at[parity, k]], sem_sc.at[b])

        def process_block(blk, parity):
            @pl.when(blk + 1 < _NBLK)
            def _():
                for d in idx_load(blk + 1, 1 - parity):
                    d.start()

            def group(g, carry):
                for b in range(_NBUF):
                    @pl.when(g > 0)
                    def _():
                        scatter(parity, (g - 1) * _NBUF + b, b).wait()
                    gather(parity, g * _NBUF + b, b).start()
                for b in range(_NBUF):
                    k = g * _NBUF + b
                    gather(parity, k, b).wait()
                    scatter(parity, k, b).start(add=True)
                return carry

            lax.fori_loop(0, _NGRPB, group, 0)
            for b in range(_NBUF):
                scatter(parity, (_NGRPB - 1) * _NBUF + b, b).wait()

        def pair(p, carry):
            blk0 = 2 * p
            for d in idx_load(blk0, 0):
                d.wait()
            process_block(blk0, 0)
            for d in idx_load(blk0 + 1, 1):
                d.wait()
            process_block(blk0 + 1, 1)
            return carry

        lax.fori_loop(0, _NBLK // 2, pair, 0)
        plsc.subcore_barrier()
        pltpu.sync_copy(acc.at[pl.ds(s * ms, ms)], out_hbm.at[c, pl.ds(s * ms, ms)])

    return main_kernel(table, srcs_m, trg_m, zeros_mh)


_BN = 1000


def _tc_a(x, w1, degs_t):
    def body(x_ref, w1_ref, deg_ref, table_ref, d_ref, innorm_ref):
        dg = deg_ref[...]
        on0 = lax.rsqrt(dg[:, 0] + dg[:, 4] + 2.0)
        on1 = lax.rsqrt(dg[:, 1] + dg[:, 5] + 2.0)
        in0 = lax.rsqrt(dg[:, 2] + dg[:, 6] + 2.0)
        in1 = lax.rsqrt(dg[:, 3] + dg[:, 7] + 2.0)
        innorm_ref[...] = jnp.stack([in0, in1], axis=-1)
        xb = x_ref[0]
        p = lax.dot_general(xb, w1_ref[...],
                            (((1,), (1,)), ((), ())),
                            preferred_element_type=jnp.float32)
        p0 = p[:, :F] * on0[:, None]
        p1 = p[:, F:] * on1[:, None]
        d_ref[...] = p0 + p1
        table_ref[0] = p0[:, :H]
        table_ref[1] = p1[:, :H]
        table_ref[2] = p0[:, H:]
        table_ref[3] = p1[:, H:]

    return pl.pallas_call(
        body,
        grid=(N // _BN,),
        in_specs=[
            pl.BlockSpec((1, _BN, F), lambda i: (0, i, 0)),
            pl.BlockSpec((2 * F, F), lambda i: (0, 0)),
            pl.BlockSpec((_BN, 8), lambda i: (i, 0)),
        ],
        out_specs=[
            pl.BlockSpec((4, _BN, H), lambda i: (0, i, 0)),
            pl.BlockSpec((_BN, F), lambda i: (i, 0)),
            pl.BlockSpec((_BN, 2), lambda i: (i, 0)),
        ],
        out_shape=[
            jax.ShapeDtypeStruct((4, N, H), jnp.float32),
            jax.ShapeDtypeStruct((N, F), jnp.float32),
            jax.ShapeDtypeStruct((N, 2), jnp.float32),
        ],
    )(x, w1, degs_t)


def _tc_b(aggsc, d, innorm, bias, w2):
    def body(agg_ref, d_ref, innorm_ref, bias_ref, w2_ref, out_ref):
        agg = agg_ref[...]
        dv = d_ref[...]
        a0 = jnp.concatenate([agg[0, 0], agg[1, 0]], axis=-1) + dv
        a1 = jnp.concatenate([agg[0, 1], agg[1, 1]], axis=-1) + dv
        n0 = innorm_ref[:, 0]
        n1 = innorm_ref[:, 1]
        a0 = a0 * n0[:, None] + bias_ref[0][None, :]
        a1 = a1 * n1[:, None] + bias_ref[1][None, :]
        a0 = jnp.where(a0 >= 0, a0, 0.2 * a0)
        a1 = jnp.where(a1 >= 0, a1, 0.2 * a1)
        y = jnp.concatenate([a0, a1], axis=-1)
        out_ref[0] = lax.dot_general(y, w2_ref[...],
                                     (((1,), (1,)), ((), ())),
                                     preferred_element_type=jnp.float32)

    return pl.pallas_call(
        body,
        grid=(N // _BN,),
        in_specs=[
            pl.BlockSpec((2, 2, _BN, H), lambda i: (0, 0, i, 0)),
            pl.BlockSpec((_BN, F), lambda i: (i, 0)),
            pl.BlockSpec((_BN, 2), lambda i: (i, 0)),
            pl.BlockSpec((2, F), lambda i: (0, 0)),
            pl.BlockSpec((F, 2 * F), lambda i: (0, 0)),
        ],
        out_specs=pl.BlockSpec((1, _BN, F), lambda i: (0, i, 0)),
        out_shape=jax.ShapeDtypeStruct((1, N, F), jnp.float32),
    )(aggsc, d, innorm, bias, w2)


def kernel(x, edge_index_0, edge_index_1, W1, W2, bias):
    src = jnp.concatenate([edge_index_0[0], edge_index_1[0] + N])
    trg = jnp.concatenate([edge_index_0[1], edge_index_1[1] + N])
    src_h = src.reshape(2, NTILES, _NCKH, CH)
    trg_h = trg.reshape(2, NTILES, _NCKH, CH)
    srcs_m = jnp.stack([src, src + M]).reshape(2, NTILES, _NBLK, _IBLK, CH)
    trg_m = trg.reshape(NTILES, _NBLK, _IBLK, CH)
    zeros_m = jnp.zeros((MP,), jnp.float32)
    zeros_mh = jnp.zeros((MP, H), jnp.float32)
    degs = _sc_hist(src_h, trg_h, zeros_m)[:, :, :M]
    degs_t = degs.reshape(2, 2, 2, N).transpose(3, 0, 1, 2).reshape(N, 8)
    table4, d, innorm = _tc_a(x, W1, degs_t)
    aggsc = _sc_main(table4.reshape(2 * M, H), srcs_m, trg_m, zeros_mh)[:, :M]
    return _tc_b(aggsc.reshape(2, 2, N, H), d, innorm, bias, W2)


# speedup vs baseline: 13.4277x; 1.0773x over previous
import functools

import jax
import jax.numpy as jnp
from jax import lax
from jax.experimental import pallas as pl
from jax.experimental.pallas import tpu as pltpu
from jax.experimental.pallas import tpu_sc as plsc

N = 10000
L = 2
F = 128
H = 64
M = L * N
E2 = 640000
MP = 20480
NTILES = 16
CH = 80


_NCKH = E2 // (2 * NTILES) // CH
_KH = 8


def _sc_hist(src_h, trg_h, zeros_m):
    ms = MP // NTILES
    mesh = plsc.VectorSubcoreMesh(core_axis_name="c", subcore_axis_name="s")

    @functools.partial(
        pl.kernel,
        out_type=jax.ShapeDtypeStruct((2, 2, MP), jnp.float32),
        mesh=mesh,
        scratch_types=[
            pltpu.VMEM((_NCKH, CH), jnp.int32),
            pltpu.VMEM((_NCKH, CH), jnp.int32),
            pltpu.VMEM((CH,), jnp.float32),
            pltpu.VMEM_SHARED((MP,), jnp.float32),
            pltpu.VMEM_SHARED((MP,), jnp.float32),
            pltpu.SemaphoreType.DMA,
            pltpu.SemaphoreType.DMA,
            pltpu.SemaphoreType.DMA,
        ],
    )
    def hist_kernel(src_hbm, trg_hbm, zeros_hbm, out_hbm,
                    idx_s, idx_t, ones_v, acc_s, acc_t, sem_i, sem_a, sem_b):
        c = lax.axis_index("c")
        s = lax.axis_index("s")
        for j in range(CH // 16):
            ones_v[pl.ds(j * 16, 16)] = jnp.full((16,), 1.0, jnp.float32)
        d1 = pltpu.make_async_copy(src_hbm.at[c, s], idx_s, sem_i)
        d2 = pltpu.make_async_copy(trg_hbm.at[c, s], idx_t, sem_i)
        d1.start()
        d2.start()
        pltpu.sync_copy(zeros_hbm.at[pl.ds(s * ms, ms)], acc_s.at[pl.ds(s * ms, ms)])
        pltpu.sync_copy(zeros_hbm.at[pl.ds(s * ms, ms)], acc_t.at[pl.ds(s * ms, ms)])
        d1.wait()
        d2.wait()
        plsc.subcore_barrier()

        def fire(g):
            pltpu.make_async_copy(ones_v, acc_s.at[idx_s.at[g]], sem_a).start(add=True)
            pltpu.make_async_copy(ones_v, acc_t.at[idx_t.at[g]], sem_b).start(add=True)

        def drain(g):
            pltpu.make_async_copy(ones_v, acc_s.at[idx_s.at[g]], sem_a).wait()
            pltpu.make_async_copy(ones_v, acc_t.at[idx_t.at[g]], sem_b).wait()

        for g in range(_KH):
            fire(g)

        def body(i, carry):
            drain(i - _KH)
            fire(i)
            return carry

        lax.fori_loop(_KH, _NCKH, body, 0)
        for g in range(_KH):
            drain(g)
        plsc.subcore_barrier()
        pltpu.sync_copy(acc_s.at[pl.ds(s * ms, ms)], out_hbm.at[c, 0, pl.ds(s * ms, ms)])
        pltpu.sync_copy(acc_t.at[pl.ds(s * ms, ms)], out_hbm.at[c, 1, pl.ds(s * ms, ms)])

    return hist_kernel(src_h, trg_h, zeros_m)


_CHM = 100
_NCK = E2 // NTILES // _CHM
_NBUF = 5
_IBLK = 25
_NBLK = _NCK // _IBLK
_NGRPB = _IBLK // _NBUF


def _sc_main(table, src_m, trg_m, zeros_blk):
    ms = MP // NTILES
    mesh = plsc.VectorSubcoreMesh(core_axis_name="c", subcore_axis_name="s")

    @functools.partial(
        pl.kernel,
        out_type=jax.ShapeDtypeStruct((2, MP, H), jnp.float32),
        mesh=mesh,
        scratch_types=[
            pltpu.VMEM((2, _IBLK, _CHM), jnp.int32),
            pltpu.VMEM((2, _IBLK, _CHM), jnp.int32),
            pltpu.VMEM((_NBUF, _CHM, H), jnp.float32),
            pltpu.VMEM_SHARED((MP, H), jnp.float32),
            pltpu.SemaphoreType.DMA((2,)),
            pltpu.SemaphoreType.DMA((_NBUF,)),
            pltpu.SemaphoreType.DMA((_NBUF,)),
            pltpu.SemaphoreType.DMA,
        ],
        compiler_params=pltpu.CompilerParams(use_tc_tiling_on_sc=False),
    )
    def main_kernel(table_hbm, src_hbm, trg_hbm, zeros_hbm, out_hbm,
                    isb, itb, rows, acc, sem_ib, sem_g, sem_sc, sem_z):
        c = lax.axis_index("c")
        s = lax.axis_index("s")
        cm = c * M

        def idx_load(blk, parity):
            return (pltpu.make_async_copy(src_hbm.at[s, blk],
                                          isb.at[parity], sem_ib.at[parity]),
                    pltpu.make_async_copy(trg_hbm.at[s, blk],
                                          itb.at[parity], sem_ib.at[parity]))

        for d in idx_load(0, 0):
            d.start()
        zd = pltpu.make_async_copy(zeros_hbm, acc.at[pl.ds(s * ms, ms)], sem_z)
        zd.start()
        zd.wait()
        plsc.subcore_barrier()

        def gather(parity, k, b):
            return pltpu.make_async_copy(table_hbm.at[c].at[isb.at[parity, k]],
                                         rows.at[b], sem_g.at[b])

        def scatter(parity, k, b):
            return pltpu.make_async_copy(rows.at[b],
                                         acc.at[itb.at[parity, k]], sem_sc.at[b])

        def process_block(blk, parity):
            @pl.when(blk + 1 < _NBLK)
            def _():
                for d in idx_load(blk + 1, 1 - parity):
                    d.start()

            def group(g, carry):
                for b in range(_NBUF):
                    @pl.when(g > 0)
                    def _():
                        scatter(parity, (g - 1) * _NBUF + b, b).wait()
                    gather(parity, g * _NBUF + b, b).start()
                for b in range(_NBUF):
                    k = g * _NBUF + b
                    gather(parity, k, b).wait()
                    scatter(parity, k, b).start(add=True)
                return carry

            lax.fori_loop(0, _NGRPB, group, 0)
            for b in range(_NBUF):
                scatter(parity, (_NGRPB - 1) * _NBUF + b, b).wait()

        def pair(p, carry):
            blk0 = 2 * p
            for d in idx_load(blk0, 0):
                d.wait()
            process_block(blk0, 0)
            for d in idx_load(blk0 + 1, 1):
                d.wait()
            process_block(blk0 + 1, 1)
            return carry

        lax.fori_loop(0, _NBLK // 2, pair, 0)
        plsc.subcore_barrier()
        pltpu.sync_copy(acc.at[pl.ds(s * ms, ms)], out_hbm.at[c, pl.ds(s * ms, ms)])

    return main_kernel(table, src_m, trg_m, zeros_blk)


_BN = 1000


def _tc_a(x, w1, degs_t):
    def body(x_ref, w1_ref, deg_ref, table_ref, d_ref, innorm_ref):
        dg = deg_ref[...]
        on0 = lax.rsqrt(dg[:, 0] + dg[:, 4] + 2.0)
        on1 = lax.rsqrt(dg[:, 1] + dg[:, 5] + 2.0)
        in0 = lax.rsqrt(dg[:, 2] + dg[:, 6] + 2.0)
        in1 = lax.rsqrt(dg[:, 3] + dg[:, 7] + 2.0)
        innorm_ref[...] = jnp.stack([in0, in1], axis=-1)
        xb = x_ref[0]
        p = lax.dot_general(xb, w1_ref[...],
                            (((1,), (1,)), ((), ())),
                            preferred_element_type=jnp.float32)
        p0 = p[:, :F] * on0[:, None]
        p1 = p[:, F:] * on1[:, None]
        d_ref[...] = p0 + p1
        table_ref[0] = p0[:, :H]
        table_ref[1] = p1[:, :H]
        table_ref[2] = p0[:, H:]
        table_ref[3] = p1[:, H:]

    return pl.pallas_call(
        body,
        grid=(N // _BN,),
        in_specs=[
            pl.BlockSpec((1, _BN, F), lambda i: (0, i, 0)),
            pl.BlockSpec((2 * F, F), lambda i: (0, 0)),
            pl.BlockSpec((_BN, 8), lambda i: (i, 0)),
        ],
        out_specs=[
            pl.BlockSpec((4, _BN, H), lambda i: (0, i, 0)),
            pl.BlockSpec((_BN, F), lambda i: (i, 0)),
            pl.BlockSpec((_BN, 2), lambda i: (i, 0)),
        ],
        out_shape=[
            jax.ShapeDtypeStruct((4, N, H), jnp.float32),
            jax.ShapeDtypeStruct((N, F), jnp.float32),
            jax.ShapeDtypeStruct((N, 2), jnp.float32),
        ],
    )(x, w1, degs_t)


def _tc_b(aggsc, d, innorm, bias, w2):
    def body(agg0_ref, agg1_ref, d_ref, innorm_ref, bias_ref, w2_ref, out_ref):
        agg0 = agg0_ref[...]
        agg1 = agg1_ref[...]
        dv = d_ref[...]
        a0 = jnp.concatenate([agg0[0], agg0[1]], axis=-1) + dv
        a1 = jnp.concatenate([agg1[0], agg1[1]], axis=-1) + dv
        n0 = innorm_ref[:, 0]
        n1 = innorm_ref[:, 1]
        a0 = a0 * n0[:, None] + bias_ref[0][None, :]
        a1 = a1 * n1[:, None] + bias_ref[1][None, :]
        a0 = jnp.where(a0 >= 0, a0, 0.2 * a0)
        a1 = jnp.where(a1 >= 0, a1, 0.2 * a1)
        y = jnp.concatenate([a0, a1], axis=-1)
        out_ref[0] = lax.dot_general(y, w2_ref[...],
                                     (((1,), (1,)), ((), ())),
                                     preferred_element_type=jnp.float32)

    return pl.pallas_call(
        body,
        grid=(N // _BN,),
        in_specs=[
            pl.BlockSpec((2, _BN, H), lambda i: (0, i, 0)),
            pl.BlockSpec((2, _BN, H), lambda i: (0, N // _BN + i, 0)),
            pl.BlockSpec((_BN, F), lambda i: (i, 0)),
            pl.BlockSpec((_BN, 2), lambda i: (i, 0)),
            pl.BlockSpec((2, F), lambda i: (0, 0)),
            pl.BlockSpec((F, 2 * F), lambda i: (0, 0)),
        ],
        out_specs=pl.BlockSpec((1, _BN, F), lambda i: (0, i, 0)),
        out_shape=jax.ShapeDtypeStruct((1, N, F), jnp.float32),
    )(aggsc, aggsc, d, innorm, bias, w2)


def kernel(x, edge_index_0, edge_index_1, W1, W2, bias):
    src = jnp.concatenate([edge_index_0[0], edge_index_1[0] + N])
    trg = jnp.concatenate([edge_index_0[1], edge_index_1[1] + N])
    src_h = src.reshape(2, NTILES, _NCKH, CH)
    trg_h = trg.reshape(2, NTILES, _NCKH, CH)
    src_m = src.reshape(NTILES, _NBLK, _IBLK, _CHM)
    trg_m = trg.reshape(NTILES, _NBLK, _IBLK, _CHM)
    zeros_m = jnp.zeros((MP,), jnp.float32)
    zeros_blk = jnp.zeros((MP // NTILES, H), jnp.float32)
    degs = _sc_hist(src_h, trg_h, zeros_m)[:, :, :M]
    degs_t = degs.reshape(2, 2, 2, N).transpose(3, 0, 1, 2).reshape(N, 8)
    table4, d, innorm = _tc_a(x, W1, degs_t)
    table3 = table4.reshape(2, M, H)
    aggsc = _sc_main(table3, src_m, trg_m, zeros_blk)
    return _tc_b(aggsc, d, innorm, bias, W2)


# speedup vs baseline: 14.0345x; 1.0452x over previous
import functools

import jax
import jax.numpy as jnp
from jax import lax
from jax.experimental import pallas as pl
from jax.experimental.pallas import tpu as pltpu
from jax.experimental.pallas import tpu_sc as plsc

N = 10000
L = 2
F = 128
H = 64
M = L * N
E2 = 640000
MP = 20480
NTILES = 16
CH = 80


_NCKH = E2 // (2 * NTILES) // CH
_KH = 8


def _sc_hist(src_h, trg_h, zeros_m):
    ms = MP // NTILES
    mesh = plsc.VectorSubcoreMesh(core_axis_name="c", subcore_axis_name="s")

    @functools.partial(
        pl.kernel,
        out_type=jax.ShapeDtypeStruct((2, 2, MP), jnp.float32),
        mesh=mesh,
        scratch_types=[
            pltpu.VMEM((_NCKH, CH), jnp.int32),
            pltpu.VMEM((_NCKH, CH), jnp.int32),
            pltpu.VMEM((CH,), jnp.float32),
            pltpu.VMEM_SHARED((MP,), jnp.float32),
            pltpu.VMEM_SHARED((MP,), jnp.float32),
            pltpu.SemaphoreType.DMA,
            pltpu.SemaphoreType.DMA,
            pltpu.SemaphoreType.DMA,
        ],
    )
    def hist_kernel(src_hbm, trg_hbm, zeros_hbm, out_hbm,
                    idx_s, idx_t, ones_v, acc_s, acc_t, sem_i, sem_a, sem_b):
        c = lax.axis_index("c")
        s = lax.axis_index("s")
        for j in range(CH // 16):
            ones_v[pl.ds(j * 16, 16)] = jnp.full((16,), 1.0, jnp.float32)
        d1 = pltpu.make_async_copy(src_hbm.at[c, s], idx_s, sem_i)
        d2 = pltpu.make_async_copy(trg_hbm.at[c, s], idx_t, sem_i)
        d1.start()
        d2.start()
        pltpu.sync_copy(zeros_hbm.at[pl.ds(s * ms, ms)], acc_s.at[pl.ds(s * ms, ms)])
        pltpu.sync_copy(zeros_hbm.at[pl.ds(s * ms, ms)], acc_t.at[pl.ds(s * ms, ms)])
        d1.wait()
        d2.wait()
        plsc.subcore_barrier()

        def fire(g):
            pltpu.make_async_copy(ones_v, acc_s.at[idx_s.at[g]], sem_a).start(add=True)
            pltpu.make_async_copy(ones_v, acc_t.at[idx_t.at[g]], sem_b).start(add=True)

        def drain(g):
            pltpu.make_async_copy(ones_v, acc_s.at[idx_s.at[g]], sem_a).wait()
            pltpu.make_async_copy(ones_v, acc_t.at[idx_t.at[g]], sem_b).wait()

        for g in range(_KH):
            fire(g)

        def body(i, carry):
            drain(i - _KH)
            fire(i)
            return carry

        lax.fori_loop(_KH, _NCKH, body, 0)
        for g in range(_KH):
            drain(g)
        plsc.subcore_barrier()
        pltpu.sync_copy(acc_s.at[pl.ds(s * ms, ms)], out_hbm.at[c, 0, pl.ds(s * ms, ms)])
        pltpu.sync_copy(acc_t.at[pl.ds(s * ms, ms)], out_hbm.at[c, 1, pl.ds(s * ms, ms)])

    return hist_kernel(src_h, trg_h, zeros_m)


_CHM = 125
_NCK = E2 // NTILES // _CHM
_NBUF = 5
_IBLK = 10
_NBLK = _NCK // _IBLK
_NGRPB = _IBLK // _NBUF


def _sc_main(table, src_m, trg_m, zeros_blk):
    ms = MP // NTILES
    mesh = plsc.VectorSubcoreMesh(core_axis_name="c", subcore_axis_name="s")

    @functools.partial(
        pl.kernel,
        out_type=jax.ShapeDtypeStruct((MP, F), jnp.float32),
        mesh=mesh,
        scratch_types=[
            pltpu.VMEM((2, _IBLK, _CHM), jnp.int32),
            pltpu.VMEM((2, _IBLK, _CHM), jnp.int32),
            pltpu.VMEM((_NBUF, _CHM, H), jnp.float32),
            pltpu.VMEM_SHARED((MP, H), jnp.float32),
            pltpu.SemaphoreType.DMA((2,)),
            pltpu.SemaphoreType.DMA((_NBUF,)),
            pltpu.SemaphoreType.DMA((_NBUF,)),
            pltpu.SemaphoreType.DMA,
        ],
        compiler_params=pltpu.CompilerParams(use_tc_tiling_on_sc=False),
    )
    def main_kernel(table_hbm, src_hbm, trg_hbm, zeros_hbm, out_hbm,
                    isb, itb, rows, acc, sem_ib, sem_g, sem_sc, sem_z):
        c = lax.axis_index("c")
        s = lax.axis_index("s")
        cm = c * M

        def idx_load(blk, parity):
            return (pltpu.make_async_copy(src_hbm.at[s, blk],
                                          isb.at[parity], sem_ib.at[parity]),
                    pltpu.make_async_copy(trg_hbm.at[s, blk],
                                          itb.at[parity], sem_ib.at[parity]))

        for d in idx_load(0, 0):
            d.start()
        zd = pltpu.make_async_copy(zeros_hbm, acc.at[pl.ds(s * ms, ms)], sem_z)
        zd.start()
        zd.wait()
        plsc.subcore_barrier()

        def gather(parity, k, b):
            return pltpu.make_async_copy(table_hbm.at[c].at[isb.at[parity, k]],
                                         rows.at[b], sem_g.at[b])

        def scatter(parity, k, b):
            return pltpu.make_async_copy(rows.at[b],
                                         acc.at[itb.at[parity, k]], sem_sc.at[b])

        def process_block(blk, parity):
            @pl.when(blk + 1 < _NBLK)
            def _():
                for d in idx_load(blk + 1, 1 - parity):
                    d.start()

            def group(g, carry):
                for b in range(_NBUF):
                    @pl.when(g > 0)
                    def _():
                        scatter(parity, (g - 1) * _NBUF + b, b).wait()
                    gather(parity, g * _NBUF + b, b).start()
                for b in range(_NBUF):
                    k = g * _NBUF + b
                    gather(parity, k, b).wait()
                    scatter(parity, k, b).start(add=True)
                return carry

            lax.fori_loop(0, _NGRPB, group, 0)
            for b in range(_NBUF):
                scatter(parity, (_NGRPB - 1) * _NBUF + b, b).wait()

        def pair(p, carry):
            blk0 = 2 * p
            for d in idx_load(blk0, 0):
                d.wait()
            process_block(blk0, 0)
            for d in idx_load(blk0 + 1, 1):
                d.wait()
            process_block(blk0 + 1, 1)
            return carry

        lax.fori_loop(0, _NBLK // 2, pair, 0)
        plsc.subcore_barrier()
        pltpu.sync_copy(acc.at[pl.ds(s * ms, ms)],
                        out_hbm.at[pl.ds(s * ms, ms), pl.ds(c * H, H)])

    return main_kernel(table, src_m, trg_m, zeros_blk)


_BN = 1000


def _tc_a(x, w1, degs_t):
    def body(x_ref, w1_ref, deg_ref, table_ref, d_ref, innorm_ref):
        dg = deg_ref[...]
        on0 = lax.rsqrt(dg[:, 0] + dg[:, 4] + 2.0)
        on1 = lax.rsqrt(dg[:, 1] + dg[:, 5] + 2.0)
        in0 = lax.rsqrt(dg[:, 2] + dg[:, 6] + 2.0)
        in1 = lax.rsqrt(dg[:, 3] + dg[:, 7] + 2.0)
        innorm_ref[...] = jnp.stack([in0, in1], axis=-1)
        xb = x_ref[0]
        p = lax.dot_general(xb, w1_ref[...],
                            (((1,), (1,)), ((), ())),
                            preferred_element_type=jnp.float32)
        p0 = p[:, :F] * on0[:, None]
        p1 = p[:, F:] * on1[:, None]
        d_ref[...] = p0 + p1
        table_ref[0] = p0[:, :H]
        table_ref[1] = p1[:, :H]
        table_ref[2] = p0[:, H:]
        table_ref[3] = p1[:, H:]

    return pl.pallas_call(
        body,
        grid=(N // _BN,),
        in_specs=[
            pl.BlockSpec((1, _BN, F), lambda i: (0, i, 0)),
            pl.BlockSpec((2 * F, F), lambda i: (0, 0)),
            pl.BlockSpec((_BN, 8), lambda i: (i, 0)),
        ],
        out_specs=[
            pl.BlockSpec((4, _BN, H), lambda i: (0, i, 0)),
            pl.BlockSpec((_BN, F), lambda i: (i, 0)),
            pl.BlockSpec((_BN, 2), lambda i: (i, 0)),
        ],
        out_shape=[
            jax.ShapeDtypeStruct((4, N, H), jnp.float32),
            jax.ShapeDtypeStruct((N, F), jnp.float32),
            jax.ShapeDtypeStruct((N, 2), jnp.float32),
        ],
    )(x, w1, degs_t)


def _tc_b(aggsc, d, innorm, bias, w2):
    def body(agg0_ref, agg1_ref, d_ref, innorm_ref, bias_ref, w2_ref, out_ref):
        dv = d_ref[...]
        a0 = agg0_ref[...] + dv
        a1 = agg1_ref[...] + dv
        n0 = innorm_ref[:, 0]
        n1 = innorm_ref[:, 1]
        a0 = a0 * n0[:, None] + bias_ref[0][None, :]
        a1 = a1 * n1[:, None] + bias_ref[1][None, :]
        a0 = jnp.where(a0 >= 0, a0, 0.2 * a0)
        a1 = jnp.where(a1 >= 0, a1, 0.2 * a1)
        y = jnp.concatenate([a0, a1], axis=-1)
        out_ref[0] = lax.dot_general(y, w2_ref[...],
                                     (((1,), (1,)), ((), ())),
                                     preferred_element_type=jnp.float32)

    return pl.pallas_call(
        body,
        grid=(N // _BN,),
        in_specs=[
            pl.BlockSpec((_BN, F), lambda i: (i, 0)),
            pl.BlockSpec((_BN, F), lambda i: (N // _BN + i, 0)),
            pl.BlockSpec((_BN, F), lambda i: (i, 0)),
            pl.BlockSpec((_BN, 2), lambda i: (i, 0)),
            pl.BlockSpec((2, F), lambda i: (0, 0)),
            pl.BlockSpec((F, 2 * F), lambda i: (0, 0)),
        ],
        out_specs=pl.BlockSpec((1, _BN, F), lambda i: (0, i, 0)),
        out_shape=jax.ShapeDtypeStruct((1, N, F), jnp.float32),
    )(aggsc, aggsc, d, innorm, bias, w2)


def kernel(x, edge_index_0, edge_index_1, W1, W2, bias):
    src = jnp.concatenate([edge_index_0[0], edge_index_1[0] + N])
    trg = jnp.concatenate([edge_index_0[1], edge_index_1[1] + N])
    src_h = src.reshape(2, NTILES, _NCKH, CH)
    trg_h = trg.reshape(2, NTILES, _NCKH, CH)
    src_m = src.reshape(NTILES, _NBLK, _IBLK, _CHM)
    trg_m = trg.reshape(NTILES, _NBLK, _IBLK, _CHM)
    zeros_m = jnp.zeros((MP,), jnp.float32)
    zeros_blk = jnp.zeros((MP // NTILES, H), jnp.float32)
    degs = _sc_hist(src_h, trg_h, zeros_m)[:, :, :M]
    degs_t = degs.reshape(2, 2, 2, N).transpose(3, 0, 1, 2).reshape(N, 8)
    table4, d, innorm = _tc_a(x, W1, degs_t)
    table3 = table4.reshape(2, M, H)
    aggsc = _sc_main(table3, src_m, trg_m, zeros_blk)
    return _tc_b(aggsc, d, innorm, bias, W2)
